# baseline (speedup 1.0000x reference)
import functools

import jax
import jax.numpy as jnp
from jax import lax
from jax.experimental import pallas as pl
from jax.experimental.pallas import tpu as pltpu
from jax.experimental.pallas import tpu_sc as plsc

_NCORES = 2
_NSUB = 16
_NW = _NCORES * _NSUB
_CSZ = 128
_K = 10


def _tc_pack_table(table_t):
    d, v = table_t.shape
    wpr = d // 2
    sec = 128 // wpr
    lblk = 2048
    nblk = -(-v // (sec * lblk))

    def body(*refs):
        o_ref = refs[-1]
        xs = jnp.concatenate(
            [refs[u][...][:wpr] for u in range(sec)]
            + [refs[u][...][wpr:] for u in range(sec)],
            axis=0,
        )
        t = jnp.transpose(xs)
        zlo = lax.bitcast_convert_type(t[:, :128], jnp.int32)
        zhi = lax.bitcast_convert_type(t[:, 128:], jnp.int32)
        rlo = ((zlo + 0x7FFF + ((zlo >> 16) & 1)) >> 16) & 0xFFFF
        rhi = ((zhi + 0x7FFF + ((zhi >> 16) & 1)) >> 16) & 0xFFFF
        o_ref[...] = rlo | (rhi << 16)

    maxblk = (v - 1) // lblk

    return pl.pallas_call(
        body,
        grid=(nblk,),
        in_specs=[
            pl.BlockSpec(
                (d, lblk),
                functools.partial(
                    lambda u, i: (0, jnp.minimum(i * sec + u, maxblk)), u
                ),
            )
            for u in range(sec)
        ],
        out_specs=pl.BlockSpec((lblk, 128), lambda i: (i, 0)),
        out_shape=jax.ShapeDtypeStruct((nblk * lblk, 128), jnp.int32),
    )(*([table_t] * sec))


def _sc_gather(idx, table):
    nw, chunks, csz = idx.shape
    _, d = table.shape
    per_w = chunks * csz
    n = nw * per_w
    groups = chunks // _K
    half = groups // 2
    mesh = plsc.VectorSubcoreMesh(core_axis_name="c", subcore_axis_name="s")

    @functools.partial(
        pl.kernel,
        out_type=jax.ShapeDtypeStruct((n, d), table.dtype),
        mesh=mesh,
        compiler_params=pltpu.CompilerParams(use_tc_tiling_on_sc=False, skip_device_barrier=True),
        scratch_types=[
            pltpu.VMEM((chunks, csz), jnp.int32),
            pltpu.VMEM((2 * _K, csz, d), table.dtype),
            pltpu.SemaphoreType.DMA,
            pltpu.SemaphoreType.DMA,
        ],
    )
    def gather_kernel(idx_hbm, table_hbm, out_hbm, idx_v, rows_v, gsem, wsem):
        wid = lax.axis_index("s") * _NCORES + lax.axis_index("c")
        base = wid * per_w
        pltpu.sync_copy(idx_hbm.at[wid], idx_v)

        def issue_gathers(g, setoff):
            for b in range(_K):
                pltpu.async_copy(
                    table_hbm.at[idx_v.at[g * _K + b]],
                    rows_v.at[setoff + b],
                    gsem,
                )

        def drain_g(setoff):
            for b in range(_K):
                pltpu.make_async_copy(
                    table_hbm.at[pl.ds(0, csz)], rows_v.at[setoff + b], gsem
                ).wait()

        def issue_wb(g, setoff):
            for b in range(_K):
                pltpu.async_copy(
                    rows_v.at[setoff + b],
                    out_hbm.at[pl.ds(base + (g * _K + b) * csz, csz)],
                    wsem,
                )

        def drain_wb(setoff):
            for b in range(_K):
                pltpu.make_async_copy(
                    rows_v.at[setoff + b], out_hbm.at[pl.ds(0, csz)], wsem
                ).wait()

        issue_gathers(0, 0)

        def body(h, carry):
            ge = 2 * h
            go = 2 * h + 1
            drain_g(0)
            issue_wb(ge, 0)

            @pl.when(h >= 1)
            def _():
                drain_wb(_K)

            issue_gathers(go, _K)
            drain_wb(0)
            @pl.when(h + 1 < half)
            def _():
                issue_gathers(ge + 2, 0)

            drain_g(_K)
            issue_wb(go, _K)
            return carry

        lax.fori_loop(0, half, body, 0)
        drain_wb(_K)

    return gather_kernel(idx, table)


def _tc_head(x128, wlo, whi, b, bsz, nt):
    nc = wlo.shape[2]
    bb = 2048
    nb = bsz // bb

    def body(x_ref, wlo_ref, whi_ref, b_ref, o_ref):
        t = pl.program_id(1)
        w = x_ref[...]
        flo = lax.bitcast_convert_type(w << 16, jnp.float32)
        fhi = lax.bitcast_convert_type(w & jnp.int32(-65536), jnp.float32)
        p = jnp.dot(jax.nn.gelu(flo), wlo_ref[0], preferred_element_type=jnp.float32)
        p += jnp.dot(jax.nn.gelu(fhi), whi_ref[0], preferred_element_type=jnp.float32)

        @pl.when(t == 0)
        def _():
            o_ref[...] = p + b_ref[...]

        @pl.when(t > 0)
        def _():
            o_ref[...] += p

    return pl.pallas_call(
        body,
        grid=(nb, nt),
        in_specs=[
            pl.BlockSpec((bb, 128), lambda i, t: (t * nb + i, 0)),
            pl.BlockSpec((1, 128, nc), lambda i, t: (t, 0, 0)),
            pl.BlockSpec((1, 128, nc), lambda i, t: (t, 0, 0)),
            pl.BlockSpec((1, nc), lambda i, t: (0, 0)),
        ],
        out_specs=pl.BlockSpec((bb, nc), lambda i, t: (i, 0)),
        out_shape=jax.ShapeDtypeStruct((bsz, nc), jnp.float32),
    )(x128, wlo, whi, b)


def kernel(x, table, W_proj, b_proj):
    bsz, s = x.shape
    v, d = table.shape
    nc = W_proj.shape[1]
    n = bsz * s
    wpr = d // 2
    sec = 128 // wpr
    lblk = 2048
    grp = sec * lblk
    upack = 128 // wpr
    nt = s // upack
    chunks = n // (_NW * _CSZ)

    table_pack = _tc_pack_table(table.T)
    vpad = table_pack.shape[0] * sec
    table_lin = table_pack.reshape(vpad, wpr)
    xt = x.astype(jnp.int32)
    xq0 = (xt // grp) * grp + (xt % lblk) * sec + (xt // lblk) % sec
    xq = xq0.reshape(bsz, nt, upack).transpose(1, 0, 2)
    idx = xq.reshape(_NW, chunks, _CSZ)
    xe = _sc_gather(idx, table_lin)
    x128 = xe.reshape(n * wpr // 128, 128)
    w4 = W_proj.reshape(nt, upack, 2, wpr, nc)
    wlo = w4[:, :, 0].reshape(nt, 128, nc)
    whi = w4[:, :, 1].reshape(nt, 128, nc)
    return _tc_head(x128, wlo, whi, b_proj.reshape(1, nc), bsz, nt)
